# Initial kernel scaffold; baseline (speedup 1.0000x reference)
import jax
import jax.numpy as jnp
from jax.experimental import pallas as pl


def kernel(x, edge_index, W1, b1, W2, b2):
    raise NotImplementedError("write your pallas kernel here")


# speedup vs baseline: 16.1022x; 16.1022x over previous
import functools

import jax
import jax.numpy as jnp
from jax import lax
from jax.experimental import pallas as pl
from jax.experimental.pallas import tpu as pltpu
from jax.experimental.pallas import tpu_sc as plsc

N = 10000
E = 320000
D_IN = 128
D_HID = 128
D_OUT = 64

CH = 128
NCH = E // CH
NC = 2
NS = 16
NW = NC * NS
CH_BASE = NCH // NW
CH_EXTRA = NCH % NW
RSTRIDE = 624
RCOPY = 128
NCOPY = 5

@functools.cache
def _mesh():
  return plsc.VectorSubcoreMesh(
      core_axis_name="c", subcore_axis_name="s", num_cores=NC, num_subcores=NS
  )


def _worker_id():
  return lax.axis_index("c") * NS + lax.axis_index("s")


def _num_chunks(w):
  return CH_BASE + jnp.where(w < CH_EXTRA, 1, 0)


@functools.cache
def _make_deg():
  @functools.partial(
      pl.kernel,
      out_type=jax.ShapeDtypeStruct((NW * N,), jnp.float32),
      mesh=_mesh(),
      scratch_types=[
          pltpu.VMEM((N,), jnp.float32),
          pltpu.VMEM((CH,), jnp.int32),
      ],
      compiler_params=pltpu.CompilerParams(needs_layout_passes=False),
  )
  def _deg(col_hbm, degp_hbm, hist, cbuf):
    w = _worker_id()

    def zero(i, carry):
      hist[pl.ds(i * 16, 16)] = jnp.zeros((16,), jnp.float32)
      return carry

    lax.fori_loop(0, N // 16, zero, 0)

    ones = jnp.ones((16,), jnp.float32)

    def body(j, carry):
      off = pl.multiple_of((w + j * NW) * CH, 8)
      pltpu.sync_copy(col_hbm.at[pl.ds(off, CH)], cbuf)

      def inner(k, c2):
        idx = cbuf[pl.ds(k * 16, 16)]
        plsc.addupdate_scatter(hist, [idx], ones)
        return c2

      return lax.fori_loop(0, CH // 16, inner, carry)

    lax.fori_loop(0, _num_chunks(w), body, 0)
    pltpu.sync_copy(hist, degp_hbm.at[pl.ds(pl.multiple_of(w * N, 8), N)])

  return _deg


@functools.cache
def _make_agg(d):
  @functools.partial(
      pl.kernel,
      out_type=jax.ShapeDtypeStruct((NC, N, d), jnp.float32),
      mesh=_mesh(),
      scratch_types=[
          pltpu.VMEM((CH, d), jnp.float32),
          pltpu.VMEM((CH,), jnp.int32),
          pltpu.VMEM((1, CH), jnp.int32),
          pltpu.VMEM_SHARED((N, d), jnp.float32),
          pltpu.SemaphoreType.DMA,
      ],
      compiler_params=pltpu.CompilerParams(needs_layout_passes=False),
  )
  def _agg(u_hbm, row_hbm, col_hbm, part_hbm, gbuf, ridx, cidx, acc, sem):
    c = lax.axis_index("c")
    s = lax.axis_index("s")
    w = c * NS + s

    def zg(r, carry):
      def zg2(k, c2):
        gbuf[r, pl.ds(k * 16, 16)] = jnp.zeros((16,), jnp.float32)
        return c2

      return lax.fori_loop(0, d // 16, zg2, carry)

    lax.fori_loop(0, CH, zg, 0)
    for k in range(NCOPY):
      r0 = pl.multiple_of(s * RSTRIDE + k * RCOPY, 8)
      pltpu.sync_copy(gbuf, acc.at[pl.ds(r0, RCOPY)])
    plsc.subcore_barrier()

    def body(j, carry):
      off = pl.multiple_of((w + j * NW) * CH, 8)
      pltpu.sync_copy(row_hbm.at[pl.ds(off, CH)], ridx)
      pltpu.sync_copy(col_hbm.at[pl.ds(off, CH)], cidx.at[0])
      pltpu.async_copy(u_hbm.at[ridx], gbuf, sem).wait()
      pltpu.sync_copy(gbuf, acc.at[cidx.at[0]], add=True)
      return carry

    lax.fori_loop(0, _num_chunks(w), body, 0)
    plsc.subcore_barrier()

    for k in range(NCOPY):
      r0 = pl.multiple_of(s * RSTRIDE + k * RCOPY, 8)
      pltpu.sync_copy(acc.at[pl.ds(r0, RCOPY)], gbuf)
      pltpu.sync_copy(gbuf, part_hbm.at[c, pl.ds(r0, RCOPY)])

  return _agg


_RB = 2000
_GRID = N // _RB


def _dinv_of(dp):
  deg = 1.0 + jnp.sum(dp[0], axis=-1)
  return lax.rsqrt(deg)[:, None]


def _u1_body(x_ref, w1_ref, dp_ref, o_ref):
  dinv = _dinv_of(dp_ref[...])
  xw = jnp.dot(x_ref[...], w1_ref[...], preferred_element_type=jnp.float32)
  o_ref[...] = xw * dinv


def _v_body(u1_ref, p0_ref, p1_ref, dp_ref, b1_ref, o_ref):
  dinv = _dinv_of(dp_ref[...])
  h = jnp.maximum(
      dinv * (u1_ref[...] + p0_ref[...] + p1_ref[...]) + b1_ref[...], 0.0
  )
  o_ref[...] = h * dinv


def _z_body(v_ref, q0_ref, q1_ref, dp_ref, b2_ref, w2_ref, o_ref):
  dinv = _dinv_of(dp_ref[...])
  t = v_ref[...] + q0_ref[...] + q1_ref[...]
  tw = jnp.dot(t, w2_ref[...], preferred_element_type=jnp.float32)
  o_ref[...] = jnp.maximum(dinv * tw + b2_ref[...], 0.0)


def _row_spec(d):
  return pl.BlockSpec((_RB, d), lambda i: (i, 0))


_full = lambda shape: pl.BlockSpec(shape, lambda i: tuple(0 for _ in shape))
_dp_spec = pl.BlockSpec((1, _RB, NW), lambda i: (i, 0, 0))

_u1_call = pl.pallas_call(
    _u1_body,
    grid=(_GRID,),
    in_specs=[_row_spec(D_IN), _full((D_IN, D_HID)), _dp_spec],
    out_specs=_row_spec(D_HID),
    out_shape=jax.ShapeDtypeStruct((N, D_HID), jnp.float32),
)

_v_call = pl.pallas_call(
    _v_body,
    grid=(_GRID,),
    in_specs=[
        _row_spec(D_HID),
        _row_spec(D_HID),
        _row_spec(D_HID),
        _dp_spec,
        _full((1, D_HID)),
    ],
    out_specs=_row_spec(D_HID),
    out_shape=jax.ShapeDtypeStruct((N, D_HID), jnp.float32),
)

_z_call = pl.pallas_call(
    _z_body,
    grid=(_GRID,),
    in_specs=[
        _row_spec(D_HID),
        _row_spec(D_HID),
        _row_spec(D_HID),
        _dp_spec,
        _full((1, D_OUT)),
        _full((D_HID, D_OUT)),
    ],
    out_specs=_row_spec(D_OUT),
    out_shape=jax.ShapeDtypeStruct((N, D_OUT), jnp.float32),
)


@jax.jit
def kernel(x, edge_index, W1, b1, W2, b2):
  row = edge_index[0]
  col = edge_index[1]
  degp = _make_deg()(col)
  dpt = degp.reshape(NW, _GRID, _RB).transpose(1, 2, 0)
  u1 = _u1_call(x, W1, dpt)
  part1 = _make_agg(D_HID)(u1, row, col)
  v = _v_call(u1, part1[0], part1[1], dpt, b1.reshape(1, D_HID))
  part2 = _make_agg(D_HID)(v, row, col)
  return _z_call(v, part2[0], part2[1], dpt, b2.reshape(1, D_OUT), W2)
